# baseline (speedup 1.0000x reference)
import jax
import jax.numpy as jnp
from jax import lax
from jax.experimental import pallas as pl
from jax.experimental.pallas import tpu as pltpu
from jax.experimental.pallas import tpu_sc as plsc

NSUB = 16
LANES = 16
ACHUNK = 96
APHASE = 4
CCHUNK = 128
NBUF = 3
PREF = 2


def _ring(nchunks, issue_gather, wait_gather, issue_scat, wait_scat):
    for b in range(PREF):
        issue_gather(b, b)

    def chunk(j, b):
        bn = (b + PREF) % NBUF
        wait_gather(j, b)
        issue_scat(j, b)

        @pl.when(j >= NBUF - PREF)
        def _():
            wait_scat(j - (NBUF - PREF), bn)

        @pl.when(j + PREF < nchunks)
        def _():
            issue_gather(j + PREF, bn)

    def it_body(it, _):
        for k in range(NBUF):
            chunk(it * NBUF + k, k)
        return 0
    lax.fori_loop(0, nchunks // NBUF, it_body, 0)
    for jj in range(nchunks - (NBUF - PREF), nchunks):
        wait_scat(jj, jj % NBUF)


def _sc_edge_pass(n_acc, half, dh):
    def body(xl_hbm, xr_hbm, src_hbm, dst_hbm,
             suml_hbm, sumr_hbm, deg_hbm,
             acc_sh, deg_sh, src_t, dst_t, r0, r1, r2, ones, z1d,
             gsem, asem, dsem):
        cid = lax.axis_index("c")
        sid = lax.axis_index("s")
        rows = [r0, r1, r2]

        def _zrow(i, _):
            def _z16(k, _):
                r0[i, pl.ds(k * LANES, LANES)] = jnp.zeros((LANES,), jnp.float32)
                return 0
            return lax.fori_loop(0, dh // LANES, _z16, 0)
        lax.fori_loop(0, ACHUNK, _zrow, 0)

        def _z1(k, _):
            z1d[pl.ds(k * LANES, LANES)] = jnp.zeros((LANES,), jnp.float32)
            return 0
        lax.fori_loop(0, (n_acc // NSUB) // LANES, _z1, 0)

        def _o1(k, _):
            ones[pl.ds(k * LANES, LANES)] = jnp.ones((LANES,), jnp.float32)
            return 0
        lax.fori_loop(0, ACHUNK // LANES, _o1, 0)

        zrows = n_acc // NSUB
        zrow0 = sid * zrows
        for i in range(zrows // ACHUNK):
            pltpu.sync_copy(r0, acc_sh.at[pl.ds(zrow0 + i * ACHUNK, ACHUNK)])
        zrem = zrows % ACHUNK
        if zrem:
            pltpu.sync_copy(
                r0.at[pl.ds(0, zrem)],
                acc_sh.at[pl.ds(zrow0 + zrows - zrem, zrem)])
        pltpu.sync_copy(z1d, deg_sh.at[pl.ds(zrow0, zrows)])
        plsc.subcore_barrier()

        def edge_loop(xh_hbm, do_deg):
            for ph in range(APHASE):
                tid = sid * APHASE + ph
                pltpu.sync_copy(src_hbm.at[tid], src_t)
                pltpu.sync_copy(dst_hbm.at[tid], dst_t)

                def issue_gather(j, b):
                    pltpu.async_copy(xh_hbm.at[src_t.at[j]], rows[b],
                                     gsem.at[b])

                def wait_gather(j, b):
                    pltpu.make_async_copy(xh_hbm.at[src_t.at[j]], rows[b],
                                          gsem.at[b]).wait()

                def issue_scat(j, b):
                    pltpu.async_copy(rows[b], acc_sh.at[dst_t.at[j]],
                                     asem.at[b], add=True)
                    if do_deg:
                        pltpu.async_copy(ones, deg_sh.at[dst_t.at[j]],
                                         dsem.at[b], add=True)

                def wait_scat(j, b):
                    pltpu.make_async_copy(rows[b], acc_sh.at[dst_t.at[j]],
                                          asem.at[b]).wait()
                    if do_deg:
                        pltpu.make_async_copy(ones, deg_sh.at[dst_t.at[j]],
                                              dsem.at[b]).wait()

                _ring(half, issue_gather, wait_gather, issue_scat, wait_scat)

        pl.when(cid == 0)(lambda: edge_loop(xl_hbm, True))
        pl.when(cid == 1)(lambda: edge_loop(xr_hbm, False))
        plsc.subcore_barrier()

        def wout(out_hbm):
            pltpu.sync_copy(acc_sh.at[pl.ds(zrow0, zrows)],
                            out_hbm.at[pl.ds(zrow0, zrows)])

        pl.when(cid == 0)(lambda: wout(suml_hbm))
        pl.when(cid == 1)(lambda: wout(sumr_hbm))
        pl.when(cid == 0)(lambda: pltpu.sync_copy(
            deg_sh.at[pl.ds(zrow0, zrows)],
            deg_hbm.at[pl.ds(zrow0, zrows)]))

    return body


def _tc_node_pass(xl_ref, xr_ref, sl_ref, sr_ref, deg_ref, w_ref,
                  q_ref, base_ref, invd_ref):
    invd = 1.0 / jnp.maximum(deg_ref[...], 1.0)
    w1l = w_ref[:, 0:128]
    w1r = w_ref[:, 128:256]
    w2l = w_ref[:, 256:384]
    w2r = w_ref[:, 384:512]
    w3l = w_ref[:, 512:640]
    w3r = w_ref[:, 640:768]
    xl = xl_ref[...]
    xr = xr_ref[...]
    ml = sl_ref[...] * invd
    mr = sr_ref[...] * invd
    dl = xl - ml
    dr = xr - mr
    q = (jnp.sum(w2l * dl * dl, axis=1, keepdims=True) +
         jnp.sum(w2r * dr * dr, axis=1, keepdims=True))
    base = (jnp.sum(w1l * ml + w3l * xl, axis=1, keepdims=True) +
            jnp.sum(w1r * mr + w3r * xr, axis=1, keepdims=True))
    q_ref[...] = q
    base_ref[...] = base
    invd_ref[...] = invd


def _sc_scalar_pass(n_acc, cptc):
    def body(q_hbm, src_hbm, dst_hbm, base_hbm, invd_hbm,
             score_hbm, s2_sh, q_sh, src_t, dst_t, v0, v1, v2, z1d,
             sv, bv, iv, gsem, ssem):
        cid = lax.axis_index("c")
        sid = lax.axis_index("s")
        rows_per_tile = n_acc // NSUB
        row0 = sid * rows_per_tile
        vals = [v0, v1, v2]

        @pl.when(cid == 0)
        def _():
            def _z1(k, _):
                z1d[pl.ds(k * LANES, LANES)] = jnp.zeros((LANES,), jnp.float32)
                return 0
            lax.fori_loop(0, rows_per_tile // LANES, _z1, 0)
            pltpu.sync_copy(z1d, s2_sh.at[pl.ds(row0, rows_per_tile)])
            @pl.when(sid == 0)
            def _():
                pltpu.sync_copy(q_hbm, q_sh)
            plsc.subcore_barrier()

            pltpu.sync_copy(src_hbm.at[sid], src_t)
            pltpu.sync_copy(dst_hbm.at[sid], dst_t)

            def issue_gather(j, b):
                pltpu.async_copy(q_sh.at[src_t.at[j]], vals[b], gsem.at[b])

            def wait_gather(j, b):
                pltpu.make_async_copy(q_sh.at[src_t.at[j]], vals[b],
                                      gsem.at[b]).wait()

            def issue_scat(j, b):
                pltpu.async_copy(vals[b], s2_sh.at[dst_t.at[j]], ssem.at[b],
                                 add=True)

            def wait_scat(j, b):
                pltpu.make_async_copy(vals[b], s2_sh.at[dst_t.at[j]],
                                      ssem.at[b]).wait()

            _ring(cptc, issue_gather, wait_gather, issue_scat, wait_scat)
            plsc.subcore_barrier()

            pltpu.sync_copy(s2_sh.at[pl.ds(row0, rows_per_tile)], sv)
            pltpu.sync_copy(base_hbm.at[pl.ds(row0, rows_per_tile)], bv)
            pltpu.sync_copy(invd_hbm.at[pl.ds(row0, rows_per_tile)], iv)

            def _node(k, _):
                sl = pl.ds(k * LANES, LANES)
                z = bv[sl] + sv[sl] * iv[sl]
                sv[sl] = 1.0 / (1.0 + jnp.exp(-z))
                return 0
            lax.fori_loop(0, rows_per_tile // LANES, _node, 0)
            pltpu.sync_copy(sv, score_hbm.at[pl.ds(row0, rows_per_tile)])

    return body


def _pad_edges(src, dst, n, e_pad, chunk):
    e = src.shape[0]
    pad = e_pad - e
    src_p = jnp.concatenate([src, jnp.zeros((pad,), jnp.int32)])
    dst_p = jnp.concatenate(
        [dst, n + (jnp.arange(pad, dtype=jnp.int32) % chunk)])
    return src_p, dst_p


@jax.jit
def kernel(x, edge_index, W):
    n, d = x.shape
    e = edge_index.shape[1]
    dh = d // 2

    half = -(-e // (NSUB * ACHUNK * APHASE * NBUF)) * NBUF
    e_pad_a = NSUB * ACHUNK * APHASE * half
    cptc = -(-e // (NSUB * CCHUNK * NBUF)) * NBUF
    e_pad_c = NSUB * CCHUNK * cptc
    n_acc = -(-(n + CCHUNK) // (NSUB * 128)) * (NSUB * 128)

    src_a, dst_a = _pad_edges(edge_index[0], edge_index[1], n, e_pad_a, ACHUNK)
    src_ra = src_a.reshape(NSUB * APHASE, half, ACHUNK)
    dst_ra = dst_a.reshape(NSUB * APHASE, half, ACHUNK)
    src_c, dst_c = _pad_edges(edge_index[0], edge_index[1], n, e_pad_c, CCHUNK)
    src_rc = src_c.reshape(NSUB, cptc, CCHUNK)
    dst_rc = dst_c.reshape(NSUB, cptc, CCHUNK)
    xl = x[:, :dh]
    xr = x[:, dh:]

    mesh = plsc.VectorSubcoreMesh(core_axis_name="c", subcore_axis_name="s")

    edge_kernel = pl.kernel(
        _sc_edge_pass(n_acc, half, dh),
        out_type=[
            jax.ShapeDtypeStruct((n_acc, dh), jnp.float32),
            jax.ShapeDtypeStruct((n_acc, dh), jnp.float32),
            jax.ShapeDtypeStruct((n_acc,), jnp.float32),
        ],
        mesh=mesh,
        scratch_types=[
            pltpu.MemorySpace.VMEM_SHARED((n_acc, dh), jnp.float32),
            pltpu.MemorySpace.VMEM_SHARED((n_acc,), jnp.float32),
            pltpu.VMEM((half, ACHUNK), jnp.int32),
            pltpu.VMEM((half, ACHUNK), jnp.int32),
            pltpu.VMEM((ACHUNK, dh), jnp.float32),
            pltpu.VMEM((ACHUNK, dh), jnp.float32),
            pltpu.VMEM((ACHUNK, dh), jnp.float32),
            pltpu.VMEM((ACHUNK,), jnp.float32),
            pltpu.VMEM((n_acc // NSUB,), jnp.float32),
            pltpu.SemaphoreType.DMA((NBUF,)),
            pltpu.SemaphoreType.DMA((NBUF,)),
            pltpu.SemaphoreType.DMA((NBUF,)),
        ],
    )
    suml, sumr, deg = edge_kernel(xl, xr, src_ra, dst_ra)

    nb = 400
    grid = n // nb
    q, base, invd = pl.pallas_call(
        _tc_node_pass,
        grid=(grid,),
        in_specs=[
            pl.BlockSpec((nb, dh), lambda i: (i, 0)),
            pl.BlockSpec((nb, dh), lambda i: (i, 0)),
            pl.BlockSpec((nb, dh), lambda i: (i, 0)),
            pl.BlockSpec((nb, dh), lambda i: (i, 0)),
            pl.BlockSpec((nb, 1), lambda i: (i, 0)),
            pl.BlockSpec((1, 3 * d), lambda i: (0, 0)),
        ],
        out_specs=[
            pl.BlockSpec((nb, 1), lambda i: (i, 0)),
            pl.BlockSpec((nb, 1), lambda i: (i, 0)),
            pl.BlockSpec((nb, 1), lambda i: (i, 0)),
        ],
        out_shape=[
            jax.ShapeDtypeStruct((n, 1), jnp.float32),
            jax.ShapeDtypeStruct((n, 1), jnp.float32),
            jax.ShapeDtypeStruct((n, 1), jnp.float32),
        ],
    )(xl, xr, suml, sumr, deg.reshape(n_acc, 1), W)

    zpad = jnp.zeros((n_acc - n,), jnp.float32)
    q_p = jnp.concatenate([q[:, 0], zpad])
    base_p = jnp.concatenate([base[:, 0], zpad])
    invd_p = jnp.concatenate([invd[:, 0], zpad])
    rows_per_tile = n_acc // NSUB
    scalar_kernel = pl.kernel(
        _sc_scalar_pass(n_acc, cptc),
        out_type=jax.ShapeDtypeStruct((n_acc,), jnp.float32),
        mesh=mesh,
        scratch_types=[
            pltpu.MemorySpace.VMEM_SHARED((n_acc,), jnp.float32),
            pltpu.MemorySpace.VMEM_SHARED((n_acc,), jnp.float32),
            pltpu.VMEM((cptc, CCHUNK), jnp.int32),
            pltpu.VMEM((cptc, CCHUNK), jnp.int32),
            pltpu.VMEM((CCHUNK,), jnp.float32),
            pltpu.VMEM((CCHUNK,), jnp.float32),
            pltpu.VMEM((CCHUNK,), jnp.float32),
            pltpu.VMEM((rows_per_tile,), jnp.float32),
            pltpu.VMEM((rows_per_tile,), jnp.float32),
            pltpu.VMEM((rows_per_tile,), jnp.float32),
            pltpu.VMEM((rows_per_tile,), jnp.float32),
            pltpu.SemaphoreType.DMA((NBUF,)),
            pltpu.SemaphoreType.DMA((NBUF,)),
        ],
    )
    score = scalar_kernel(q_p, src_rc, dst_rc, base_p, invd_p)
    return score[:n, None]


# speedup vs baseline: 2.0320x; 2.0320x over previous
import jax
import jax.numpy as jnp
from jax import lax
from jax.experimental import pallas as pl
from jax.experimental.pallas import tpu as pltpu
from jax.experimental.pallas import tpu_sc as plsc

NSUB = 16
LANES = 16
ACHUNK = 80
APHASE = 2
CCHUNK = 128
NBUF = 3
PREF = 2


def _ring(nchunks, issue_gather, wait_gather, issue_scat, wait_scat):
    for b in range(PREF):
        issue_gather(b, b)

    def chunk(j, b):
        bn = (b + PREF) % NBUF
        wait_gather(j, b)
        issue_scat(j, b)

        @pl.when(j >= NBUF - PREF)
        def _():
            wait_scat(j - (NBUF - PREF), bn)

        @pl.when(j + PREF < nchunks)
        def _():
            issue_gather(j + PREF, bn)

    def it_body(it, _):
        for k in range(NBUF):
            chunk(it * NBUF + k, k)
        return 0
    lax.fori_loop(0, nchunks // NBUF, it_body, 0)
    for jj in range(nchunks - (NBUF - PREF), nchunks):
        wait_scat(jj, jj % NBUF)


def _sc_edge_pass(n_acc, half, dh):
    def body(xl_hbm, xr_hbm, src_hbm, dst_hbm,
             suml_hbm, sumr_hbm, deg_hbm,
             acc_sh, deg_sh, src_t, dst_t, r0, r1, r2, ones, z1d,
             gsem, asem, dsem):
        cid = lax.axis_index("c")
        sid = lax.axis_index("s")
        rows = [r0, r1, r2]

        def _zrow(i, _):
            def _z16(k, _):
                r0[i, pl.ds(k * LANES, LANES)] = jnp.zeros((LANES,), jnp.float32)
                return 0
            return lax.fori_loop(0, dh // LANES, _z16, 0)
        lax.fori_loop(0, ACHUNK, _zrow, 0)

        def _z1(k, _):
            z1d[pl.ds(k * LANES, LANES)] = jnp.zeros((LANES,), jnp.float32)
            return 0
        lax.fori_loop(0, (n_acc // NSUB) // LANES, _z1, 0)

        def _o1(k, _):
            ones[pl.ds(k * LANES, LANES)] = jnp.ones((LANES,), jnp.float32)
            return 0
        lax.fori_loop(0, ACHUNK // LANES, _o1, 0)

        zrows = n_acc // NSUB
        zrow0 = sid * zrows
        for i in range(zrows // ACHUNK):
            pltpu.sync_copy(r0, acc_sh.at[pl.ds(zrow0 + i * ACHUNK, ACHUNK)])
        zrem = zrows % ACHUNK
        if zrem:
            pltpu.sync_copy(
                r0.at[pl.ds(0, zrem)],
                acc_sh.at[pl.ds(zrow0 + zrows - zrem, zrem)])
        pltpu.sync_copy(z1d, deg_sh.at[pl.ds(zrow0, zrows)])
        plsc.subcore_barrier()

        def edge_loop(xh_hbm, do_deg):
            for ph in range(APHASE):
                tid = sid * APHASE + ph
                pltpu.sync_copy(src_hbm.at[tid], src_t)
                pltpu.sync_copy(dst_hbm.at[tid], dst_t)

                def issue_gather(j, b):
                    pltpu.async_copy(xh_hbm.at[src_t.at[j]], rows[b],
                                     gsem.at[b])

                def wait_gather(j, b):
                    pltpu.make_async_copy(xh_hbm.at[src_t.at[j]], rows[b],
                                          gsem.at[b]).wait()

                def issue_scat(j, b):
                    pltpu.async_copy(rows[b], acc_sh.at[dst_t.at[j]],
                                     asem.at[b], add=True)
                    if do_deg:
                        pltpu.async_copy(ones, deg_sh.at[dst_t.at[j]],
                                         dsem.at[b], add=True)

                def wait_scat(j, b):
                    pltpu.make_async_copy(rows[b], acc_sh.at[dst_t.at[j]],
                                          asem.at[b]).wait()
                    if do_deg:
                        pltpu.make_async_copy(ones, deg_sh.at[dst_t.at[j]],
                                              dsem.at[b]).wait()

                _ring(half, issue_gather, wait_gather, issue_scat, wait_scat)

        pl.when(cid == 0)(lambda: edge_loop(xl_hbm, True))
        pl.when(cid == 1)(lambda: edge_loop(xr_hbm, False))
        plsc.subcore_barrier()

        def wout(out_hbm):
            pltpu.sync_copy(acc_sh.at[pl.ds(zrow0, zrows)],
                            out_hbm.at[pl.ds(zrow0, zrows)])

        pl.when(cid == 0)(lambda: wout(suml_hbm))
        pl.when(cid == 1)(lambda: wout(sumr_hbm))
        pl.when(cid == 0)(lambda: pltpu.sync_copy(
            deg_sh.at[pl.ds(zrow0, zrows)],
            deg_hbm.at[pl.ds(zrow0, zrows)]))

    return body


def _tc_node_pass(xl_ref, xr_ref, sl_ref, sr_ref, deg_ref, w_ref,
                  q_ref, base_ref, invd_ref):
    invd = 1.0 / jnp.maximum(deg_ref[...], 1.0)
    w1l = w_ref[:, 0:128]
    w1r = w_ref[:, 128:256]
    w2l = w_ref[:, 256:384]
    w2r = w_ref[:, 384:512]
    w3l = w_ref[:, 512:640]
    w3r = w_ref[:, 640:768]
    xl = xl_ref[...]
    xr = xr_ref[...]
    ml = sl_ref[...] * invd
    mr = sr_ref[...] * invd
    dl = xl - ml
    dr = xr - mr
    q = (jnp.sum(w2l * dl * dl, axis=1, keepdims=True) +
         jnp.sum(w2r * dr * dr, axis=1, keepdims=True))
    base = (jnp.sum(w1l * ml + w3l * xl, axis=1, keepdims=True) +
            jnp.sum(w1r * mr + w3r * xr, axis=1, keepdims=True))
    q_ref[...] = q
    base_ref[...] = base
    invd_ref[...] = invd


def _sc_scalar_pass(n_acc, cptc):
    def body(q_hbm, src_hbm, dst_hbm, base_hbm, invd_hbm,
             score_hbm, s2_sh, q_sh, src_t, dst_t, v0, v1, v2, z1d,
             sv, bv, iv, gsem, ssem):
        cid = lax.axis_index("c")
        sid = lax.axis_index("s")
        rows_per_tile = n_acc // NSUB
        row0 = sid * rows_per_tile
        vals = [v0, v1, v2]

        @pl.when(cid == 0)
        def _():
            def _z1(k, _):
                z1d[pl.ds(k * LANES, LANES)] = jnp.zeros((LANES,), jnp.float32)
                return 0
            lax.fori_loop(0, rows_per_tile // LANES, _z1, 0)
            pltpu.sync_copy(z1d, s2_sh.at[pl.ds(row0, rows_per_tile)])
            @pl.when(sid == 0)
            def _():
                pltpu.sync_copy(q_hbm, q_sh)
            plsc.subcore_barrier()

            pltpu.sync_copy(src_hbm.at[sid], src_t)
            pltpu.sync_copy(dst_hbm.at[sid], dst_t)

            def issue_gather(j, b):
                pltpu.async_copy(q_sh.at[src_t.at[j]], vals[b], gsem.at[b])

            def wait_gather(j, b):
                pltpu.make_async_copy(q_sh.at[src_t.at[j]], vals[b],
                                      gsem.at[b]).wait()

            def issue_scat(j, b):
                pltpu.async_copy(vals[b], s2_sh.at[dst_t.at[j]], ssem.at[b],
                                 add=True)

            def wait_scat(j, b):
                pltpu.make_async_copy(vals[b], s2_sh.at[dst_t.at[j]],
                                      ssem.at[b]).wait()

            _ring(cptc, issue_gather, wait_gather, issue_scat, wait_scat)
            plsc.subcore_barrier()

            pltpu.sync_copy(s2_sh.at[pl.ds(row0, rows_per_tile)], sv)
            pltpu.sync_copy(base_hbm.at[pl.ds(row0, rows_per_tile)], bv)
            pltpu.sync_copy(invd_hbm.at[pl.ds(row0, rows_per_tile)], iv)

            def _node(k, _):
                sl = pl.ds(k * LANES, LANES)
                z = bv[sl] + sv[sl] * iv[sl]
                sv[sl] = 1.0 / (1.0 + jnp.exp(-z))
                return 0
            lax.fori_loop(0, rows_per_tile // LANES, _node, 0)
            pltpu.sync_copy(sv, score_hbm.at[pl.ds(row0, rows_per_tile)])

    return body


def _pad_edges(src, dst, n, e_pad, chunk):
    e = src.shape[0]
    pad = e_pad - e
    src_p = jnp.concatenate([src, jnp.zeros((pad,), jnp.int32)])
    dst_p = jnp.concatenate(
        [dst, n + (jnp.arange(pad, dtype=jnp.int32) % chunk)])
    return src_p, dst_p


@jax.jit
def kernel(x, edge_index, W):
    n, d = x.shape
    e = edge_index.shape[1]
    dh = d // 2

    half = -(-e // (NSUB * ACHUNK * APHASE * NBUF)) * NBUF
    e_pad_a = NSUB * ACHUNK * APHASE * half
    cptc = -(-e // (NSUB * CCHUNK * NBUF)) * NBUF
    e_pad_c = NSUB * CCHUNK * cptc
    n_acc = -(-(n + CCHUNK) // (NSUB * 128)) * (NSUB * 128)

    src_a, dst_a = _pad_edges(edge_index[0], edge_index[1], n, e_pad_a, ACHUNK)
    src_ra = src_a.reshape(NSUB * APHASE, half, ACHUNK)
    dst_ra = dst_a.reshape(NSUB * APHASE, half, ACHUNK)
    src_c, dst_c = _pad_edges(edge_index[0], edge_index[1], n, e_pad_c, CCHUNK)
    src_rc = src_c.reshape(NSUB, cptc, CCHUNK)
    dst_rc = dst_c.reshape(NSUB, cptc, CCHUNK)
    xl = x[:, :dh]
    xr = x[:, dh:]

    mesh = plsc.VectorSubcoreMesh(core_axis_name="c", subcore_axis_name="s")

    edge_kernel = pl.kernel(
        _sc_edge_pass(n_acc, half, dh),
        out_type=[
            jax.ShapeDtypeStruct((n_acc, dh), jnp.float32),
            jax.ShapeDtypeStruct((n_acc, dh), jnp.float32),
            jax.ShapeDtypeStruct((n_acc,), jnp.float32),
        ],
        mesh=mesh,
        scratch_types=[
            pltpu.MemorySpace.VMEM_SHARED((n_acc, dh), jnp.float32),
            pltpu.MemorySpace.VMEM_SHARED((n_acc,), jnp.float32),
            pltpu.VMEM((half, ACHUNK), jnp.int32),
            pltpu.VMEM((half, ACHUNK), jnp.int32),
            pltpu.VMEM((ACHUNK, dh), jnp.float32),
            pltpu.VMEM((ACHUNK, dh), jnp.float32),
            pltpu.VMEM((ACHUNK, dh), jnp.float32),
            pltpu.VMEM((ACHUNK,), jnp.float32),
            pltpu.VMEM((n_acc // NSUB,), jnp.float32),
            pltpu.SemaphoreType.DMA((NBUF,)),
            pltpu.SemaphoreType.DMA((NBUF,)),
            pltpu.SemaphoreType.DMA((NBUF,)),
        ],
    )
    suml, sumr, deg = edge_kernel(xl, xr, src_ra, dst_ra)

    nb = 400
    grid = n // nb
    q, base, invd = pl.pallas_call(
        _tc_node_pass,
        grid=(grid,),
        in_specs=[
            pl.BlockSpec((nb, dh), lambda i: (i, 0)),
            pl.BlockSpec((nb, dh), lambda i: (i, 0)),
            pl.BlockSpec((nb, dh), lambda i: (i, 0)),
            pl.BlockSpec((nb, dh), lambda i: (i, 0)),
            pl.BlockSpec((nb, 1), lambda i: (i, 0)),
            pl.BlockSpec((1, 3 * d), lambda i: (0, 0)),
        ],
        out_specs=[
            pl.BlockSpec((nb, 1), lambda i: (i, 0)),
            pl.BlockSpec((nb, 1), lambda i: (i, 0)),
            pl.BlockSpec((nb, 1), lambda i: (i, 0)),
        ],
        out_shape=[
            jax.ShapeDtypeStruct((n, 1), jnp.float32),
            jax.ShapeDtypeStruct((n, 1), jnp.float32),
            jax.ShapeDtypeStruct((n, 1), jnp.float32),
        ],
    )(xl, xr, suml, sumr, deg.reshape(n_acc, 1), W)

    zpad = jnp.zeros((n_acc - n,), jnp.float32)
    q_p = jnp.concatenate([q[:, 0], zpad])
    base_p = jnp.concatenate([base[:, 0], zpad])
    invd_p = jnp.concatenate([invd[:, 0], zpad])
    rows_per_tile = n_acc // NSUB
    scalar_kernel = pl.kernel(
        _sc_scalar_pass(n_acc, cptc),
        out_type=jax.ShapeDtypeStruct((n_acc,), jnp.float32),
        mesh=mesh,
        scratch_types=[
            pltpu.MemorySpace.VMEM_SHARED((n_acc,), jnp.float32),
            pltpu.MemorySpace.VMEM_SHARED((n_acc,), jnp.float32),
            pltpu.VMEM((cptc, CCHUNK), jnp.int32),
            pltpu.VMEM((cptc, CCHUNK), jnp.int32),
            pltpu.VMEM((CCHUNK,), jnp.float32),
            pltpu.VMEM((CCHUNK,), jnp.float32),
            pltpu.VMEM((CCHUNK,), jnp.float32),
            pltpu.VMEM((rows_per_tile,), jnp.float32),
            pltpu.VMEM((rows_per_tile,), jnp.float32),
            pltpu.VMEM((rows_per_tile,), jnp.float32),
            pltpu.VMEM((rows_per_tile,), jnp.float32),
            pltpu.SemaphoreType.DMA((NBUF,)),
            pltpu.SemaphoreType.DMA((NBUF,)),
        ],
    )
    score = scalar_kernel(q_p, src_rc, dst_rc, base_p, invd_p)
    return score[:n, None]
